# baseline (speedup 1.0000x reference)
import jax
import jax.numpy as jnp
from jax.experimental import pallas as pl
from jax.experimental.pallas import tpu as pltpu

N = 1024
NFEAT = 128
NHID = 64
NHEADS = 3
ALPHA = 0.2
LOG2E = 1.4426950408889634
BLK = 1024
GRID = N // BLK


def _gat_kernel(x_ref, adj_ref, w0_ref, a0_ref, w1_ref, a1_ref, w2_ref, a2_ref,
                out_ref, hext_ref, nf1_ref, nf2_ref):
    i = pl.program_id(0)
    w_refs = (w0_ref, w1_ref, w2_ref)
    a_refs = (a0_ref, a1_ref, a2_ref)

    @pl.when(i == 0)
    def _():
        xv = x_ref[...]
        ones = jnp.ones((N, NHID), dtype=jnp.bfloat16)
        for hd in range(NHEADS):
            w = w_refs[hd][...]
            h = jnp.dot(xv, w, preferred_element_type=jnp.float32)
            hext_ref[hd, :, 0:NHID] = h.astype(jnp.bfloat16)
            hext_ref[hd, :, NHID:2 * NHID] = ones
            a1 = a_refs[hd][:, 0:NHID] * (-LOG2E)
            a2 = a_refs[hd][:, NHID:2 * NHID] * (-LOG2E)
            nf1_ref[hd] = jax.lax.dot_general(
                h, a1, (((1,), (1,)), ((), ())),
                preferred_element_type=jnp.float32)
            ht = jax.lax.dot_general(
                w, xv, (((0,), (1,)), ((), ())),
                preferred_element_type=jnp.float32)
            nf2_ref[hd] = jnp.dot(a2, ht, preferred_element_type=jnp.float32)

    adjb = adj_ref[...]
    rows = jax.lax.broadcasted_iota(jnp.int32, (BLK, N), 0) + i * BLK
    cols = jax.lax.broadcasted_iota(jnp.int32, (BLK, N), 1)
    mask = (adjb != 0.0) | (rows == cols)

    for hd in range(NHEADS):
        nf1b = nf1_ref[hd, pl.ds(i * BLK, BLK), :]
        nf2r = nf2_ref[hd]
        t = nf1b + nf2r
        g = jnp.exp2(jnp.minimum(t, ALPHA * t))
        e = jnp.where(mask, g, 0.0).astype(jnp.bfloat16)
        hp = jnp.dot(e, hext_ref[hd], preferred_element_type=jnp.float32)
        v = hp[:, 0:NHID] / hp[:, NHID:NHID + 1]
        out_ref[:, hd * NHID:(hd + 1) * NHID] = jnp.where(
            v > 0.0, v, jnp.exp(jnp.minimum(v, 0.0)) - 1.0)


def kernel(x, adj, W0, a0, W1, a1, W2, a2):
    full = lambda shape: pl.BlockSpec(shape, lambda i: tuple(0 for _ in shape))
    wspec = full((NFEAT, NHID))
    aspec = full((1, 2 * NHID))
    return pl.pallas_call(
        _gat_kernel,
        grid=(GRID,),
        in_specs=[
            full((N, NFEAT)),
            pl.BlockSpec((BLK, N), lambda i: (i, 0)),
            wspec, aspec, wspec, aspec, wspec, aspec,
        ],
        out_specs=pl.BlockSpec((BLK, NHEADS * NHID), lambda i: (i, 0)),
        out_shape=jax.ShapeDtypeStruct((N, NHEADS * NHID), jnp.float32),
        scratch_shapes=[
            pltpu.VMEM((NHEADS, N, 2 * NHID), jnp.bfloat16),
            pltpu.VMEM((NHEADS, N, 1), jnp.float32),
            pltpu.VMEM((NHEADS, 1, N), jnp.float32),
        ],
    )(x, adj, W0, a0, W1, a1, W2, a2)


# speedup vs baseline: 1.0162x; 1.0162x over previous
import jax
import jax.numpy as jnp
from jax.experimental import pallas as pl
from jax.experimental.pallas import tpu as pltpu

N = 1024
NFEAT = 128
NHID = 64
NHEADS = 3
ALPHA = 0.2
LOG2E = 1.4426950408889634
BLK = 512
GRID = N // BLK


def _gat_kernel(x_ref, adj_ref, w0_ref, a0_ref, w1_ref, a1_ref, w2_ref, a2_ref,
                out_ref, hext_ref, nf1_ref, nf2_ref):
    i = pl.program_id(0)
    w_refs = (w0_ref, w1_ref, w2_ref)
    a_refs = (a0_ref, a1_ref, a2_ref)

    @pl.when(i == 0)
    def _():
        xv = x_ref[...]
        ones = jnp.ones((N, NHID), dtype=jnp.bfloat16)
        for hd in range(NHEADS):
            w = w_refs[hd][...]
            h = jnp.dot(xv, w, preferred_element_type=jnp.float32)
            hext_ref[hd, :, 0:NHID] = h.astype(jnp.bfloat16)
            hext_ref[hd, :, NHID:2 * NHID] = ones
            a1 = a_refs[hd][:, 0:NHID] * (-LOG2E)
            a2 = a_refs[hd][:, NHID:2 * NHID] * (-LOG2E)
            nf1_ref[hd] = jax.lax.dot_general(
                h, a1, (((1,), (1,)), ((), ())),
                preferred_element_type=jnp.float32)
            ht = jax.lax.dot_general(
                w, xv, (((0,), (1,)), ((), ())),
                preferred_element_type=jnp.float32)
            nf2_ref[hd] = jnp.dot(a2, ht, preferred_element_type=jnp.float32)

    adjb = adj_ref[...]
    rows = jax.lax.broadcasted_iota(jnp.int32, (BLK, N), 0) + i * BLK
    cols = jax.lax.broadcasted_iota(jnp.int32, (BLK, N), 1)
    mask = (adjb != 0.0) | (rows == cols)

    for hd in range(NHEADS):
        nf1b = nf1_ref[hd, pl.ds(i * BLK, BLK), :]
        nf2r = nf2_ref[hd]
        t = nf1b + nf2r
        g = jnp.exp2(jnp.minimum(t, ALPHA * t))
        e = jnp.where(mask, g, 0.0).astype(jnp.bfloat16)
        hp = jnp.dot(e, hext_ref[hd], preferred_element_type=jnp.float32)
        v = hp[:, 0:NHID] / hp[:, NHID:NHID + 1]
        out_ref[:, hd * NHID:(hd + 1) * NHID] = jnp.maximum(
            v, jnp.exp(jnp.minimum(v, 0.0)) - 1.0)


def kernel(x, adj, W0, a0, W1, a1, W2, a2):
    full = lambda shape: pl.BlockSpec(shape, lambda i: tuple(0 for _ in shape))
    wspec = full((NFEAT, NHID))
    aspec = full((1, 2 * NHID))
    return pl.pallas_call(
        _gat_kernel,
        grid=(GRID,),
        in_specs=[
            full((N, NFEAT)),
            pl.BlockSpec((BLK, N), lambda i: (i, 0)),
            wspec, aspec, wspec, aspec, wspec, aspec,
        ],
        out_specs=pl.BlockSpec((BLK, NHEADS * NHID), lambda i: (i, 0)),
        out_shape=jax.ShapeDtypeStruct((N, NHEADS * NHID), jnp.float32),
        scratch_shapes=[
            pltpu.VMEM((NHEADS, N, 2 * NHID), jnp.bfloat16),
            pltpu.VMEM((NHEADS, N, 1), jnp.float32),
            pltpu.VMEM((NHEADS, 1, N), jnp.float32),
        ],
    )(x, adj, W0, a0, W1, a1, W2, a2)
